# baseline (speedup 1.0000x reference)
import functools

import jax
import jax.numpy as jnp
from jax import lax
from jax.experimental import pallas as pl
from jax.experimental.pallas import tpu as pltpu
from jax.experimental.pallas import tpu_sc as plsc

NC = 2
NS = 16
NW = NC * NS
B = 16384
D = 32
BPW = B // NW
NPAIR = BPW // 16

_mesh = plsc.VectorSubcoreMesh(core_axis_name="c", subcore_axis_name="s")


@functools.partial(
    pl.kernel,
    mesh=_mesh,
    out_type=jax.ShapeDtypeStruct((B,), jnp.float32),
    scratch_types=[
        pltpu.VMEM((BPW,), jnp.int32),
        pltpu.VMEM((BPW,), jnp.int32),
        pltpu.VMEM((8, D, 128), jnp.float32),
        pltpu.VMEM((8, D, 128), jnp.float32),
        pltpu.VMEM((48,), jnp.float32),
        pltpu.VMEM((BPW,), jnp.float32),
        pltpu.SemaphoreType.DMA,
        pltpu.SemaphoreType.DMA,
    ],
    compiler_params=pltpu.CompilerParams(
        needs_layout_passes=False, use_tc_tiling_on_sc=True),
)
def _gmf_sc(user_hbm, item_hbm, ut_hbm, it_hbm, wb_hbm, out_hbm,
            idx_uv, idx_iv, blk_u, blk_i, wb_v, out_v, sem_u, sem_i):
    wid = lax.axis_index("s") * NC + lax.axis_index("c")
    base = wid * BPW

    pltpu.sync_copy(wb_hbm, wb_v)
    pltpu.sync_copy(user_hbm.at[pl.ds(base, BPW)], idx_uv)
    pltpu.sync_copy(item_hbm.at[pl.ds(base, BPW)], idx_iv)

    lane = jnp.arange(16, dtype=jnp.int32)
    half = lane < 8
    jv = lane & 7
    bv = wb_v[pl.ds(32, 16)]
    w_lo = wb_v[pl.ds(0, 16)]
    w_hi = wb_v[pl.ds(16, 16)]

    def pair_body(p, _):
        e0 = p * 16
        ru16 = idx_uv[pl.ds(e0, 16)]
        ri16 = idx_iv[pl.ds(e0, 16)]
        cbu = (ru16 >> 7) << 7
        cbi = (ri16 >> 7) << 7
        col_u = ru16 & 127
        col_i = ri16 & 127
        accs = [None, None]
        for h in range(2):
            copies = []
            for j in range(8):
                cu = pl.multiple_of(cbu[h * 8 + j], 128)
                copies.append(pltpu.async_copy(
                    ut_hbm.at[:, pl.ds(cu, 128)], blk_u.at[j], sem_u))
                ci = pl.multiple_of(cbi[h * 8 + j], 128)
                copies.append(pltpu.async_copy(
                    it_hbm.at[:, pl.ds(ci, 128)], blk_i.at[j], sem_i))
            for cp in copies:
                cp.wait()
            pacc = [jnp.zeros((16,), jnp.float32) for _ in range(4)]
            for d in range(D):
                dv = jnp.full((16,), d, jnp.int32)
                u = plsc.load_gather(blk_u, [jv, dv, col_u])
                iv = plsc.load_gather(blk_i, [jv, dv, col_i])
                w_d = (w_lo if d < 16 else w_hi)[d % 16]
                pacc[d % 4] = pacc[d % 4] + (u * iv) * w_d
            accs[h] = (pacc[0] + pacc[1]) + (pacc[2] + pacc[3])
        s = jnp.where(half, accs[0], accs[1]) + bv
        out_v[pl.ds(e0, 16)] = 1.0 / (1.0 + jnp.exp(-s))
        return 0

    lax.fori_loop(0, NPAIR, pair_body, 0)

    pltpu.sync_copy(out_v, out_hbm.at[pl.ds(base, BPW)])


def kernel(user, item, user_table, item_table, W, b):
    wb = jnp.concatenate(
        [W.reshape(-1), jnp.broadcast_to(b.reshape(-1), (16,))]).astype(jnp.float32)
    return _gmf_sc(user.astype(jnp.int32), item.astype(jnp.int32),
                   user_table.T, item_table.T, wb)
